# baseline (speedup 1.0000x reference)
import functools

import jax
import jax.numpy as jnp
from jax import lax
from jax.experimental import pallas as pl
from jax.experimental.pallas import tpu as pltpu
from jax.experimental.pallas import tpu_sc as plsc

_TOP_K = 2
_ROUTED_SCALING = 1.0
_BF = 1024

_T = 32
_E = 16


def _router_tc_kernel(x_ref, rw_ref, bias_ref, scores_ref, biased_ref):
    logits = jax.lax.dot_general(
        x_ref[...], rw_ref[...], (((1,), (1,)), ((), ())),
        preferred_element_type=jnp.float32)
    m = jnp.max(logits, axis=1, keepdims=True)
    ex = jnp.exp(logits - m)
    scores = ex / jnp.sum(ex, axis=1, keepdims=True)
    scores_ref[...] = scores
    biased_ref[...] = scores + bias_ref[...]


def _router_sc_body(scores_hbm, biased_hbm, out_hbm, sc_v, bi_v, comb_v, red_v):
    nc = 2
    t = lax.axis_index("s") * nc + lax.axis_index("c")
    pltpu.sync_copy(scores_hbm.at[pl.ds(t * _E, _E)], sc_v)
    pltpu.sync_copy(biased_hbm.at[pl.ds(t * _E, _E)], bi_v)

    lane = lax.iota(jnp.int32, 16)

    def _allred(v, op):
        for sh in (1, 2, 4, 8):
            red_v[...] = v
            v = op(v, plsc.load_gather(red_v, [lane ^ sh]))
        return v

    scores = sc_v[...]
    biased = bi_v[...]
    lane_f = lane.astype(jnp.float32)
    m1 = _allred(biased, jnp.maximum)
    i1 = _allred(jnp.where(biased == m1, lane_f, 16.0), jnp.minimum)
    sel1 = lane_f == i1
    masked = jnp.where(sel1, -jnp.inf, biased)
    m2 = _allred(masked, jnp.maximum)
    i2 = _allred(jnp.where(masked == m2, lane_f, 16.0), jnp.minimum)
    sel2 = lane_f == i2
    comb_v[...] = jnp.where(sel1 | sel2, scores, 0.0) * _ROUTED_SCALING
    pltpu.sync_copy(comb_v, out_hbm.at[pl.ds(t * _E, _E)])


def _router_sc(scores_flat, biased_flat):
    mesh = plsc.VectorSubcoreMesh(core_axis_name="c", subcore_axis_name="s")
    fn = functools.partial(
        pl.kernel,
        out_type=jax.ShapeDtypeStruct((_T * _E,), jnp.float32),
        mesh=mesh,
        scratch_types=[
            pltpu.VMEM((_E,), jnp.float32),
            pltpu.VMEM((_E,), jnp.float32),
            pltpu.VMEM((_E,), jnp.float32),
            pltpu.VMEM((16,), jnp.float32),
        ],
        compiler_params=pltpu.CompilerParams(needs_layout_passes=False),
    )(_router_sc_body)
    return fn(scores_flat, biased_flat)


def _moe_tc_kernel(x_ref, comb_ref, wg_ref, wu_ref, wd_ref, out_ref):
    e = pl.program_id(0)
    f = pl.program_id(1)
    T, E = comb_ref.shape

    @pl.when((e == 0) & (f == 0))
    def _init():
        out_ref[...] = jnp.zeros_like(out_ref)

    x = x_ref[...]
    xg = jnp.dot(x, wg_ref[0], preferred_element_type=jnp.float32)
    xu = jnp.dot(x, wu_ref[0], preferred_element_type=jnp.float32)
    h = (xg * jax.nn.sigmoid(xg)) * xu
    eidx = jax.lax.broadcasted_iota(jnp.int32, (T, E), 1)
    ccol = jnp.sum(jnp.where(eidx == e, comb_ref[...], 0.0),
                   axis=1, keepdims=True)
    out_ref[...] += jnp.dot(h * ccol, wd_ref[0],
                            preferred_element_type=jnp.float32)


def kernel(hidden_states, router_w, correction_bias, w_gate, w_up, w_down):
    T, H = hidden_states.shape
    E, _, I = w_gate.shape
    nf = I // _BF
    bias2d = correction_bias.reshape(1, E)

    scores, biased = pl.pallas_call(
        _router_tc_kernel,
        in_specs=[
            pl.BlockSpec((T, H), lambda: (0, 0)),
            pl.BlockSpec((E, H), lambda: (0, 0)),
            pl.BlockSpec((1, E), lambda: (0, 0)),
        ],
        out_specs=[
            pl.BlockSpec((T, E), lambda: (0, 0)),
            pl.BlockSpec((T, E), lambda: (0, 0)),
        ],
        out_shape=[
            jax.ShapeDtypeStruct((T, E), jnp.float32),
            jax.ShapeDtypeStruct((T, E), jnp.float32),
        ],
    )(hidden_states, router_w, bias2d)

    combine = _router_sc(scores.reshape(T * E), biased.reshape(T * E))
    combine = combine.reshape(T, E)

    return pl.pallas_call(
        _moe_tc_kernel,
        grid=(E, nf),
        in_specs=[
            pl.BlockSpec((T, H), lambda e, f: (0, 0)),
            pl.BlockSpec((T, E), lambda e, f: (0, 0)),
            pl.BlockSpec((1, H, _BF), lambda e, f: (e, 0, f)),
            pl.BlockSpec((1, H, _BF), lambda e, f: (e, 0, f)),
            pl.BlockSpec((1, _BF, H), lambda e, f: (e, f, 0)),
        ],
        out_specs=pl.BlockSpec((T, H), lambda e, f: (0, 0)),
        out_shape=jax.ShapeDtypeStruct((T, H), jnp.float32),
        compiler_params=pltpu.CompilerParams(
            dimension_semantics=("arbitrary", "arbitrary"),
        ),
    )(hidden_states, combine, w_gate, w_up, w_down)
